# baseline (speedup 1.0000x reference)
import functools

import jax
import jax.numpy as jnp
from jax.experimental import pallas as pl
from jax.experimental.pallas import tpu as pltpu

VOCAB = 100000
D = 1024
B = 4 * 4096
R = 16


def _body(idx_ref, *refs):
    ins = refs[:R]
    out = refs[R]
    for k in range(R):
        out[k] = ins[k][0]


def _in_map(i, idx_ref, k):
    return (idx_ref[i * R + k], 0, 0)


_grid_spec = pltpu.PrefetchScalarGridSpec(
    num_scalar_prefetch=1,
    grid=(B // R,),
    in_specs=[
        pl.BlockSpec((1, 8, 128), functools.partial(_in_map, k=k)) for k in range(R)
    ],
    out_specs=pl.BlockSpec((R, 8, 128), lambda i, idx_ref: (i, 0, 0)),
)

_tc_gather = pl.pallas_call(
    _body,
    grid_spec=_grid_spec,
    out_shape=jax.ShapeDtypeStruct((B, 8, 128), jnp.float32),
)


@jax.jit
def kernel(tokens, table):
    flat = tokens.reshape(B)
    tab3 = table.reshape(VOCAB, 8, 128)
    out = _tc_gather(flat, *([tab3] * R))
    return out.reshape(tokens.shape + (D,))


# speedup vs baseline: 13.9899x; 13.9899x over previous
import functools

import jax
import jax.numpy as jnp
from jax import lax
from jax.experimental import pallas as pl
from jax.experimental.pallas import tpu as pltpu
from jax.experimental.pallas import tpu_sc as plsc

VOCAB = 100000
D = 1024
B = 4 * 4096
NC, NS = 2, 16
NW = NC * NS
B_PER_W = B // NW
CHUNK = 16
NCHUNK = B_PER_W // CHUNK
NB = 6
GDEPTH = 4

_mesh = plsc.VectorSubcoreMesh(core_axis_name="c", subcore_axis_name="s")


@functools.partial(
    pl.kernel,
    mesh=_mesh,
    out_type=jax.ShapeDtypeStruct((B, D), jnp.float32),
    scratch_types=[
        pltpu.VMEM((NCHUNK, CHUNK), jnp.int32),
    ] + [pltpu.VMEM((CHUNK, D), jnp.float32)] * NB
      + [pltpu.SemaphoreType.DMA] * (2 * NB),
)
def _embed_sc(tokens_hbm, table_hbm, out_hbm, idx_v, *bufs_and_sems):
    rows = bufs_and_sems[:NB]
    gsems = bufs_and_sems[NB:2 * NB]
    wsems = bufs_and_sems[2 * NB:]
    wid = lax.axis_index("s") * NC + lax.axis_index("c")
    base = wid * B_PER_W
    pltpu.sync_copy(tokens_hbm.at[wid], idx_v)
    gathers = [None] * NCHUNK
    writes = [None] * NCHUNK
    for b in range(GDEPTH):
        gathers[b] = pltpu.async_copy(table_hbm.at[idx_v.at[b]],
                                      rows[b % NB], gsems[b % NB])
    for j in range(NCHUNK):
        b = j % NB
        nxt = j + GDEPTH
        if nxt < NCHUNK:
            nb = nxt % NB
            if nxt - NB >= 0:
                writes[nxt - NB].wait()
            gathers[nxt] = pltpu.async_copy(table_hbm.at[idx_v.at[nxt]],
                                            rows[nb], gsems[nb])
        gathers[j].wait()
        writes[j] = pltpu.async_copy(
            rows[b], out_hbm.at[pl.ds(base + j * CHUNK, CHUNK)], wsems[b])
    for j in range(NCHUNK - NB, NCHUNK):
        writes[j].wait()


@jax.jit
def kernel(tokens, table):
    toks = tokens.reshape(NW, NCHUNK, CHUNK)
    out = _embed_sc(toks, table)
    return out.reshape(tokens.shape + (D,))
